# Initial kernel scaffold; baseline (speedup 1.0000x reference)
import jax
import jax.numpy as jnp
from jax.experimental import pallas as pl


def kernel(x, edge_index, edge_weight, batch_vec, W_conv, b_conv, W_pred, b_pred):
    raise NotImplementedError("write your pallas kernel here")


# speedup vs baseline: 4.7482x; 4.7482x over previous
import functools
import jax
import jax.numpy as jnp
from jax import lax
from jax.experimental import pallas as pl
from jax.experimental.pallas import tpu as pltpu
from jax.experimental.pallas import tpu_sc as plsc

L = 16
NC = 2
NS = 16
NW = NC * NS
K = 80
WB = 400


def _sc_edge_agg(x, src, dst, w):
    N, D = x.shape
    E = src.shape[0]
    ept = E // NW
    nchunk = ept // K
    nz = N // K
    nz_rounds = -(-nz // NS)
    nwb = N // WB
    nwb_rounds = -(-nwb // NS)
    mesh = plsc.VectorSubcoreMesh(core_axis_name="c", subcore_axis_name="s")

    @functools.partial(
        pl.kernel,
        mesh=mesh,
        out_type=jax.ShapeDtypeStruct((NC, N, D), jnp.float32),
        scratch_types=[
            pltpu.VMEM((K,), jnp.int32),
            pltpu.VMEM((K,), jnp.int32),
            pltpu.VMEM((K,), jnp.float32),
            pltpu.VMEM((K, D), jnp.float32),
            pltpu.VMEM_SHARED((N, D), jnp.float32),
            pltpu.SemaphoreType.DMA,
        ],
    )
    def k(x_hbm, src_hbm, dst_hbm, w_hbm, out_hbm,
          src_v, dst_v, w_v, rows_v, acc_sh, sem):
        cid = lax.axis_index("c")
        sid = lax.axis_index("s")
        wid = cid * NS + sid

        def zrow(i, c):
            for j in range(D // L):
                rows_v[i, pl.ds(j * L, L)] = jnp.zeros((L,), jnp.float32)
            return c
        lax.fori_loop(0, K, zrow, 0)

        for r in range(nz_rounds):
            zid = sid + r * NS

            @pl.when(zid < nz)
            def _():
                pltpu.sync_copy(rows_v, acc_sh.at[pl.ds(zid * K, K)])
        plsc.subcore_barrier()

        ebase = wid * ept

        def chunk(i, c):
            off = ebase + i * K
            pltpu.sync_copy(src_hbm.at[pl.ds(off, K)], src_v)
            pltpu.sync_copy(dst_hbm.at[pl.ds(off, K)], dst_v)
            pltpu.sync_copy(w_hbm.at[pl.ds(off, K)], w_v)
            pltpu.async_copy(x_hbm.at[src_v], rows_v, sem).wait()

            bcast_dnums = lax.GatherDimensionNumbers(
                offset_dims=(), collapsed_slice_dims=(0,), start_index_map=(0,))

            def wgroup(j, c2):
                w16 = w_v[pl.ds(j * L, L)]
                for l in range(L):
                    wb = lax.gather(w16, jnp.full((L, 1), l, jnp.int32),
                                    bcast_dnums, slice_sizes=(1,),
                                    mode=lax.GatherScatterMode.PROMISE_IN_BOUNDS)
                    row = j * L + l
                    for d in range(D // L):
                        sl = pl.ds(d * L, L)
                        rows_v[row, sl] = rows_v[row, sl] * wb
                return c2
            lax.fori_loop(0, K // L, wgroup, 0)

            pltpu.sync_copy(rows_v, acc_sh.at[dst_v], add=True)
            return c
        lax.fori_loop(0, nchunk, chunk, 0)
        plsc.subcore_barrier()

        for r in range(nwb_rounds):
            wid_chunk = sid + r * NS

            @pl.when(wid_chunk < nwb)
            def _():
                off = wid_chunk * WB
                pltpu.sync_copy(acc_sh.at[pl.ds(off, WB)],
                                out_hbm.at[cid, pl.ds(off, WB)])

    return k(x, src, dst, w)


def _tc_head(agg2, bvT, Wc, bc, Wp, bp, interpret=False):
    _, N, D = agg2.shape
    G = 128
    C = Wp.shape[1]

    def body(a_ref, bv_ref, wc_ref, bc_ref, wp_ref, bp_ref, o_ref):
        agg = a_ref[0] + a_ref[1]
        emb = jnp.dot(agg, wc_ref[...], preferred_element_type=jnp.float32)
        emb = jnp.maximum(emb + bc_ref[...], 0.0)
        oh = (bv_ref[...] == lax.broadcasted_iota(jnp.int32, (G, N), 0))
        gmat = jnp.dot(oh.astype(jnp.float32), emb,
                       preferred_element_type=jnp.float32)
        o_ref[...] = jnp.dot(gmat, wp_ref[...],
                             preferred_element_type=jnp.float32) + bp_ref[...]

    return pl.pallas_call(
        body,
        out_shape=jax.ShapeDtypeStruct((G, C), jnp.float32),
        interpret=interpret,
    )(agg2, bvT, Wc, bc, Wp, bp)


def kernel(x, edge_index, edge_weight, batch_vec, W_conv, b_conv, W_pred, b_pred):
    src = edge_index[0]
    dst = edge_index[1]
    agg2 = _sc_edge_agg(x, src, dst, edge_weight)
    return _tc_head(agg2,
                    batch_vec.reshape(1, -1).astype(jnp.int32),
                    W_conv,
                    b_conv.reshape(1, -1),
                    W_pred,
                    b_pred.reshape(1, -1))
